# baseline (speedup 1.0000x reference)
import functools

import jax
import jax.numpy as jnp
from jax import lax
from jax.experimental import pallas as pl
from jax.experimental.pallas import tpu as pltpu
from jax.experimental.pallas import tpu_sc as plsc

_ROWS = 128
_COLS = 32768
_W = 16
_RIGHT0 = 16384
_NV = 1024.0
_SLAB = 128
_NWORKERS = 32
_RPW = _ROWS // _NWORKERS
_NBUF = 3


def _mix_copy_body(x_ref, y_ref, a_v, b_v, *rest):
    bufs, isems, osems = rest[:_NBUF], rest[_NBUF:2 * _NBUF], rest[2 * _NBUF:]
    wid = lax.axis_index("s") * 2 + lax.axis_index("c")
    r0 = wid * _RPW
    rows = pl.ds(r0, _RPW)

    def in_copy(c):
        return pltpu.make_async_copy(
            x_ref.at[0, pl.ds(r0 + c, 1), :], bufs[c % _NBUF],
            isems[c % _NBUF])

    def out_copy(c):
        return pltpu.make_async_copy(
            bufs[c % _NBUF], y_ref.at[pl.ds(r0 + c, 1), :],
            osems[c % _NBUF])

    for c in range(_NBUF):
        in_copy(c).start()
    pltpu.sync_copy(x_ref.at[0, rows, pl.ds(0, _SLAB)], a_v)
    pltpu.sync_copy(x_ref.at[0, rows, pl.ds(_RIGHT0, _SLAB)], b_v)
    for i in range(_RPW):
        a = a_v[i, pl.ds(0, _W)]
        b = b_v[i, pl.ds(0, _W)]
        a_v[i, pl.ds(0, _W)] = jnp.mod(a + b, _NV) + 1.0
        b_v[i, pl.ds(0, _W)] = jnp.mod(_NV + a - b, _NV) + 1.0

    for c in range(_RPW):
        in_copy(c).wait()
        buf = bufs[c % _NBUF]
        buf[0, pl.ds(0, _W)] = a_v[c, pl.ds(0, _W)]
        buf[0, pl.ds(_RIGHT0, _W)] = b_v[c, pl.ds(0, _W)]
        out_copy(c).start()
        if c + _NBUF < _RPW:
            out_copy(c).wait()
            in_copy(c + _NBUF).start()
    for c in range(max(0, _RPW - _NBUF), _RPW):
        out_copy(c).wait()


_mix_copy = functools.partial(
    pl.kernel,
    out_type=jax.ShapeDtypeStruct((_ROWS, _COLS), jnp.float32),
    mesh=plsc.VectorSubcoreMesh(core_axis_name="c", subcore_axis_name="s"),
    compiler_params=pltpu.CompilerParams(vmem_limit_bytes=1024 * 1024),
    scratch_types=(
        [pltpu.VMEM((_RPW, _SLAB), jnp.float32)] * 2
        + [pltpu.VMEM((1, _COLS), jnp.float32)] * _NBUF
        + [pltpu.SemaphoreType.DMA] * (2 * _NBUF)
    ),
)(_mix_copy_body)


def _fix_body(x_ref, y_ref, a_v, b_v, sems_a, sems_b):
    wid = lax.axis_index("s") * 2 + lax.axis_index("c")
    rows = pl.ds(wid * _RPW, _RPW)
    ca = pltpu.make_async_copy(x_ref.at[0, rows, pl.ds(0, _SLAB)], a_v, sems_a)
    cb = pltpu.make_async_copy(
        x_ref.at[0, rows, pl.ds(_RIGHT0, _SLAB)], b_v, sems_b)
    ca.start()
    cb.start()
    ca.wait()
    cb.wait()
    for i in range(_RPW):
        a = a_v[i, pl.ds(0, _W)]
        b = b_v[i, pl.ds(0, _W)]
        a_v[i, pl.ds(0, _W)] = jnp.mod(a + b, _NV) + 1.0
        b_v[i, pl.ds(0, _W)] = jnp.mod(_NV + a - b, _NV) + 1.0
    ca2 = pltpu.make_async_copy(a_v, y_ref.at[rows, pl.ds(0, _SLAB)], sems_a)
    cb2 = pltpu.make_async_copy(
        b_v, y_ref.at[rows, pl.ds(_RIGHT0, _SLAB)], sems_b)
    ca2.start()
    cb2.start()
    ca2.wait()
    cb2.wait()


_fix = functools.partial(
    pl.kernel,
    mesh=plsc.VectorSubcoreMesh(core_axis_name="c", subcore_axis_name="s"),
    compiler_params=pltpu.CompilerParams(vmem_limit_bytes=1024 * 1024),
    scratch_types=(
        [pltpu.VMEM((_RPW, _SLAB), jnp.float32)] * 2
        + [pltpu.SemaphoreType.DMA] * 2
    ),
)(_fix_body)


def kernel(x):
    y0 = jax.new_ref(x[0])
    _fix(x, y0)
    return (y0[...], x[1], x[2])


# speedup vs baseline: 1.0003x; 1.0003x over previous
import functools

import jax
import jax.numpy as jnp
from jax import lax
from jax.experimental import pallas as pl
from jax.experimental.pallas import tpu as pltpu
from jax.experimental.pallas import tpu_sc as plsc

_ROWS = 128
_COLS = 32768
_W = 16
_RIGHT0 = 16384
_NV = 1024.0
_SLAB = 128
_NWORKERS = 32
_RPW = _ROWS // _NWORKERS
_NBUF = 3


def _mix_copy_body(x_ref, y_ref, a_v, b_v, *rest):
    bufs, isems, osems = rest[:_NBUF], rest[_NBUF:2 * _NBUF], rest[2 * _NBUF:]
    wid = lax.axis_index("s") * 2 + lax.axis_index("c")
    r0 = wid * _RPW
    rows = pl.ds(r0, _RPW)

    def in_copy(c):
        return pltpu.make_async_copy(
            x_ref.at[0, pl.ds(r0 + c, 1), :], bufs[c % _NBUF],
            isems[c % _NBUF])

    def out_copy(c):
        return pltpu.make_async_copy(
            bufs[c % _NBUF], y_ref.at[pl.ds(r0 + c, 1), :],
            osems[c % _NBUF])

    for c in range(_NBUF):
        in_copy(c).start()
    pltpu.sync_copy(x_ref.at[0, rows, pl.ds(0, _SLAB)], a_v)
    pltpu.sync_copy(x_ref.at[0, rows, pl.ds(_RIGHT0, _SLAB)], b_v)
    for i in range(_RPW):
        a = a_v[i, pl.ds(0, _W)]
        b = b_v[i, pl.ds(0, _W)]
        a_v[i, pl.ds(0, _W)] = jnp.mod(a + b, _NV) + 1.0
        b_v[i, pl.ds(0, _W)] = jnp.mod(_NV + a - b, _NV) + 1.0

    for c in range(_RPW):
        in_copy(c).wait()
        buf = bufs[c % _NBUF]
        buf[0, pl.ds(0, _W)] = a_v[c, pl.ds(0, _W)]
        buf[0, pl.ds(_RIGHT0, _W)] = b_v[c, pl.ds(0, _W)]
        out_copy(c).start()
        if c + _NBUF < _RPW:
            out_copy(c).wait()
            in_copy(c + _NBUF).start()
    for c in range(max(0, _RPW - _NBUF), _RPW):
        out_copy(c).wait()


_mix_copy = functools.partial(
    pl.kernel,
    out_type=jax.ShapeDtypeStruct((_ROWS, _COLS), jnp.float32),
    mesh=plsc.VectorSubcoreMesh(core_axis_name="c", subcore_axis_name="s"),
    compiler_params=pltpu.CompilerParams(vmem_limit_bytes=1024 * 1024),
    scratch_types=(
        [pltpu.VMEM((_RPW, _SLAB), jnp.float32)] * 2
        + [pltpu.VMEM((1, _COLS), jnp.float32)] * _NBUF
        + [pltpu.SemaphoreType.DMA] * (2 * _NBUF)
    ),
)(_mix_copy_body)


def _fix_body(x_ref, y_ref, a_v, b_v, sems_a, sems_b):
    wid = lax.axis_index("s") * 2 + lax.axis_index("c")
    rows = pl.ds(wid * _RPW, _RPW)
    ca = pltpu.make_async_copy(x_ref.at[0, rows, pl.ds(0, _SLAB)], a_v, sems_a)
    cb = pltpu.make_async_copy(
        x_ref.at[0, rows, pl.ds(_RIGHT0, _SLAB)], b_v, sems_b)
    ca.start()
    cb.start()
    ca.wait()
    cb.wait()
    for i in range(_RPW):
        a = a_v[i, pl.ds(0, _W)]
        b = b_v[i, pl.ds(0, _W)]
        a_v[i, pl.ds(0, _W)] = jnp.mod(a + b, _NV) + 1.0
        b_v[i, pl.ds(0, _W)] = jnp.mod(_NV + a - b, _NV) + 1.0
    ca2 = pltpu.make_async_copy(a_v, y_ref.at[rows, pl.ds(0, _SLAB)], sems_a)
    cb2 = pltpu.make_async_copy(
        b_v, y_ref.at[rows, pl.ds(_RIGHT0, _SLAB)], sems_b)
    ca2.start()
    cb2.start()
    ca2.wait()
    cb2.wait()


_fix = functools.partial(
    pl.kernel,
    mesh=plsc.VectorSubcoreMesh(core_axis_name="c", subcore_axis_name="s"),
    compiler_params=pltpu.CompilerParams(
        vmem_limit_bytes=1024 * 1024, skip_device_barrier=True),
    scratch_types=(
        [pltpu.VMEM((_RPW, _SLAB), jnp.float32)] * 2
        + [pltpu.SemaphoreType.DMA] * 2
    ),
)(_fix_body)


def kernel(x):
    y0 = jax.new_ref(x[0])
    _fix(x, y0)
    return (y0[...], x[1], x[2])


# speedup vs baseline: 1.0210x; 1.0208x over previous
import functools

import jax
import jax.numpy as jnp
from jax import lax
from jax.experimental import pallas as pl
from jax.experimental.pallas import tpu as pltpu
from jax.experimental.pallas import tpu_sc as plsc

_ROWS = 128
_COLS = 32768
_W = 16
_RIGHT0 = 16384
_NV = 1024.0
_SLAB = 128
_NWORKERS = 32
_RPW = _ROWS // _NWORKERS
_NBUF = 3


def _mix_copy_body(x_ref, y_ref, a_v, b_v, *rest):
    bufs, isems, osems = rest[:_NBUF], rest[_NBUF:2 * _NBUF], rest[2 * _NBUF:]
    wid = lax.axis_index("s") * 2 + lax.axis_index("c")
    r0 = wid * _RPW
    rows = pl.ds(r0, _RPW)

    def in_copy(c):
        return pltpu.make_async_copy(
            x_ref.at[0, pl.ds(r0 + c, 1), :], bufs[c % _NBUF],
            isems[c % _NBUF])

    def out_copy(c):
        return pltpu.make_async_copy(
            bufs[c % _NBUF], y_ref.at[pl.ds(r0 + c, 1), :],
            osems[c % _NBUF])

    for c in range(_NBUF):
        in_copy(c).start()
    pltpu.sync_copy(x_ref.at[0, rows, pl.ds(0, _SLAB)], a_v)
    pltpu.sync_copy(x_ref.at[0, rows, pl.ds(_RIGHT0, _SLAB)], b_v)
    for i in range(_RPW):
        a = a_v[i, pl.ds(0, _W)]
        b = b_v[i, pl.ds(0, _W)]
        a_v[i, pl.ds(0, _W)] = jnp.mod(a + b, _NV) + 1.0
        b_v[i, pl.ds(0, _W)] = jnp.mod(_NV + a - b, _NV) + 1.0

    for c in range(_RPW):
        in_copy(c).wait()
        buf = bufs[c % _NBUF]
        buf[0, pl.ds(0, _W)] = a_v[c, pl.ds(0, _W)]
        buf[0, pl.ds(_RIGHT0, _W)] = b_v[c, pl.ds(0, _W)]
        out_copy(c).start()
        if c + _NBUF < _RPW:
            out_copy(c).wait()
            in_copy(c + _NBUF).start()
    for c in range(max(0, _RPW - _NBUF), _RPW):
        out_copy(c).wait()


_mix_copy = functools.partial(
    pl.kernel,
    out_type=jax.ShapeDtypeStruct((_ROWS, _COLS), jnp.float32),
    mesh=plsc.VectorSubcoreMesh(core_axis_name="c", subcore_axis_name="s"),
    scratch_types=(
        [pltpu.VMEM((_RPW, _SLAB), jnp.float32)] * 2
        + [pltpu.VMEM((1, _COLS), jnp.float32)] * _NBUF
        + [pltpu.SemaphoreType.DMA] * (2 * _NBUF)
    ),
)(_mix_copy_body)


def kernel(x):
    y0 = _mix_copy(x)
    return (y0, x[1], x[2])
